# Initial kernel scaffold; baseline (speedup 1.0000x reference)
import jax
import jax.numpy as jnp
from jax.experimental import pallas as pl


def kernel(x, edge_index, num_nodes, params):
    raise NotImplementedError("write your pallas kernel here")


# speedup vs baseline: 3.3930x; 3.3930x over previous
import functools

import jax
import jax.numpy as jnp
from jax import lax
from jax.experimental import pallas as pl
from jax.experimental.pallas import tpu as pltpu
from jax.experimental.pallas import tpu_sc as plsc

N_NODES = 10000
H = 128
NC, NS = 2, 16
NW = NC * NS
GROUP = 128
ZCHUNK = 80
NZCHUNKS = N_NODES // ZCHUNK

_mesh = plsc.VectorSubcoreMesh(
    core_axis_name="c", subcore_axis_name="s", num_cores=NC, num_subcores=NS)


def _layer_norm(h, g, b):
    mu = jnp.mean(h, axis=-1, keepdims=True)
    var = jnp.mean((h - mu) ** 2, axis=-1, keepdims=True)
    return (h - mu) * lax.rsqrt(var + 1e-5) * g + b


def _dot(a, b):
    return jnp.dot(a, b, preferred_element_type=jnp.float32)


def _edge_mlp_body(x_ref, w1, b1, w2, b2, w3, b3, w4, b4, g, b, out_ref):
    h = x_ref[...]
    h = jnp.maximum(_dot(h, w1[...]) + b1[...], 0.0)
    h = jnp.maximum(_dot(h, w2[...]) + b2[...], 0.0)
    h = jnp.maximum(_dot(h, w3[...]) + b3[...], 0.0)
    h = _dot(h, w4[...]) + b4[...]
    out_ref[...] = _layer_norm(h, g[...], b[...])


def _node_mlp_body(p_ref, w1, b1, w2, b2, w3, b3, w4, b4, g, b, w1a, w1b,
                   node_ref, a_ref, b_ref):
    h = p_ref[0] + p_ref[1]
    h = jnp.maximum(_dot(h, w1[...]) + b1[...], 0.0)
    h = jnp.maximum(_dot(h, w2[...]) + b2[...], 0.0)
    h = jnp.maximum(_dot(h, w3[...]) + b3[...], 0.0)
    h = _dot(h, w4[...]) + b4[...]
    xn = _layer_norm(h, g[...], b[...])
    node_ref[...] = xn
    a_ref[...] = _dot(xn, w1a[...])
    b_ref[...] = _dot(xn, w1b[...])


def _final_mlp_body(g_ref, b1, w2, b2, w3, b3, w4, b4, g, b, out_ref):
    h = jnp.maximum(g_ref[...] + b1[...], 0.0)
    h = jnp.maximum(_dot(h, w2[...]) + b2[...], 0.0)
    h = jnp.maximum(_dot(h, w3[...]) + b3[...], 0.0)
    h = _dot(h, w4[...]) + b4[...]
    out_ref[...] = _layer_norm(h, g[...], b[...])


def _full(shape):
    return pl.BlockSpec(shape, lambda i: (0,) * len(shape))


def _rows(block_rows, ncols):
    return pl.BlockSpec((block_rows, ncols), lambda i: (i, 0))


def _scatter_add(x_edge, src, dst, zeros):
    E = x_edge.shape[0]
    ngroups = E // GROUP

    @functools.partial(
        pl.kernel,
        out_type=jax.ShapeDtypeStruct((NC, N_NODES, H), jnp.float32),
        mesh=_mesh,
        scratch_types=[
            pltpu.VMEM_SHARED((N_NODES, H), jnp.float32),
            pltpu.VMEM((GROUP, H), jnp.float32),
            pltpu.VMEM((GROUP,), jnp.int32),
            pltpu.VMEM((GROUP,), jnp.int32),
            pltpu.VMEM((ZCHUNK, H), jnp.float32),
        ],
    )
    def sc_scatter(xe_hbm, src_hbm, dst_hbm, zero_hbm, out_hbm,
                   acc, vrows, vsrc, vdst, zbuf):
        c = lax.axis_index("c")
        s = lax.axis_index("s")
        wid = s * NC + c

        @pl.loop(s, NZCHUNKS, step=NS)
        def _zero(j):
            off = pl.multiple_of(j * ZCHUNK, ZCHUNK)
            pltpu.sync_copy(zero_hbm.at[pl.ds(off, ZCHUNK)], zbuf)
            pltpu.sync_copy(zbuf, acc.at[pl.ds(off, ZCHUNK)])

        plsc.subcore_barrier()

        @pl.loop(wid, ngroups, step=NW)
        def _body(gi):
            base = pl.multiple_of(gi * GROUP, GROUP)
            pltpu.sync_copy(xe_hbm.at[pl.ds(base, GROUP)], vrows)
            pltpu.sync_copy(src_hbm.at[pl.ds(base, GROUP)], vsrc)
            pltpu.sync_copy(dst_hbm.at[pl.ds(base, GROUP)], vdst)
            pltpu.sync_copy(vrows, acc.at[vsrc], add=True)
            pltpu.sync_copy(vrows, acc.at[vdst], add=True)

        plsc.subcore_barrier()

        @pl.loop(s, NZCHUNKS, step=NS)
        def _flush(j):
            off = pl.multiple_of(j * ZCHUNK, ZCHUNK)
            pltpu.sync_copy(acc.at[pl.ds(off, ZCHUNK)], zbuf)
            pltpu.sync_copy(zbuf, out_hbm.at[c, pl.ds(off, ZCHUNK)])

    return sc_scatter(x_edge, src, dst, zeros)


def _gather_sum(a_tab, b_tab, src, dst):
    E = src.shape[0]
    ngroups = E // GROUP

    @functools.partial(
        pl.kernel,
        out_type=jax.ShapeDtypeStruct((E, H), jnp.float32),
        mesh=_mesh,
        scratch_types=[
            pltpu.VMEM((GROUP, H), jnp.float32),
            pltpu.VMEM((GROUP,), jnp.int32),
            pltpu.VMEM((GROUP,), jnp.int32),
        ],
    )
    def sc_gather(a_hbm, b_hbm, src_hbm, dst_hbm, out_hbm, vrows, vsrc, vdst):
        c = lax.axis_index("c")
        s = lax.axis_index("s")
        wid = s * NC + c

        @pl.loop(wid, ngroups, step=NW)
        def _body(gi):
            base = pl.multiple_of(gi * GROUP, GROUP)
            pltpu.sync_copy(src_hbm.at[pl.ds(base, GROUP)], vsrc)
            pltpu.sync_copy(dst_hbm.at[pl.ds(base, GROUP)], vdst)
            pltpu.sync_copy(a_hbm.at[vsrc], vrows)
            pltpu.sync_copy(b_hbm.at[vdst], vrows, add=True)
            pltpu.sync_copy(vrows, out_hbm.at[pl.ds(base, GROUP)])

    return sc_gather(a_tab, b_tab, src, dst)


def _unpack(p):
    (l1, l2, l3, l4), (g, b) = p
    flat = []
    for w, bb in (l1, l2, l3, l4):
        flat += [w, bb.reshape(1, -1)]
    flat += [g.reshape(1, -1), b.reshape(1, -1)]
    return flat


def kernel(x, edge_index, num_nodes, params):
    E = x.shape[0]
    D = x.shape[1]
    src = edge_index[0]
    dst = edge_index[1]

    eb = _unpack(params["eb"])
    nb = _unpack(params["nb"])
    e1 = _unpack(params["eb1"])
    w1_cat = e1[0]
    w1a, w1b = w1_cat[:H], w1_cat[H:]

    blk_e = 3200
    grid_e = E // blk_e

    x_edge = pl.pallas_call(
        _edge_mlp_body,
        grid=(grid_e,),
        in_specs=[_rows(blk_e, D), _full((D, H)), _full((1, H)),
                  _full((H, H)), _full((1, H)), _full((H, H)), _full((1, H)),
                  _full((H, H)), _full((1, H)), _full((1, H)), _full((1, H))],
        out_specs=_rows(blk_e, H),
        out_shape=jax.ShapeDtypeStruct((E, H), jnp.float32),
    )(x, *eb)

    zeros = jnp.zeros((N_NODES, H), jnp.float32)
    partials = _scatter_add(x_edge, src, dst, zeros)

    blk_n = 1000
    grid_n = N_NODES // blk_n
    x_node, a_tab, b_tab = pl.pallas_call(
        _node_mlp_body,
        grid=(grid_n,),
        in_specs=[pl.BlockSpec((NC, blk_n, H), lambda i: (0, i, 0)),
                  _full((H, H)), _full((1, H)), _full((H, H)), _full((1, H)),
                  _full((H, H)), _full((1, H)), _full((H, H)), _full((1, H)),
                  _full((1, H)), _full((1, H)), _full((H, H)), _full((H, H))],
        out_specs=[_rows(blk_n, H), _rows(blk_n, H), _rows(blk_n, H)],
        out_shape=[jax.ShapeDtypeStruct((N_NODES, H), jnp.float32),
                   jax.ShapeDtypeStruct((N_NODES, H), jnp.float32),
                   jax.ShapeDtypeStruct((N_NODES, H), jnp.float32)],
    )(partials, *nb, w1a, w1b)

    g_sum = _gather_sum(a_tab, b_tab, src, dst)

    x_edge_out = pl.pallas_call(
        _final_mlp_body,
        grid=(grid_e,),
        in_specs=[_rows(blk_e, H), _full((1, H)),
                  _full((H, H)), _full((1, H)), _full((H, H)), _full((1, H)),
                  _full((H, H)), _full((1, H)), _full((1, H)), _full((1, H))],
        out_specs=_rows(blk_e, H),
        out_shape=jax.ShapeDtypeStruct((E, H), jnp.float32),
    )(g_sum, e1[1], *e1[2:8], e1[8], e1[9])

    return (x_node, x_edge_out)
